# baseline (speedup 1.0000x reference)
import functools

import jax
import jax.numpy as jnp
from jax import lax
from jax.experimental import pallas as pl
from jax.experimental.pallas import tpu as pltpu
from jax.experimental.pallas import tpu_sc as plsc


def _build_sc_kernel(N, TD, LD, NSETS, MAXLEX, NW, C):
    NL = NSETS * MAXLEX
    IDX_W = 128
    IDX_ROWS = (C * NL) // IDX_W
    assert C * NL % IDX_W == 0
    per_w = N // NW
    assert per_w % (2 * C) == 0
    n_pairs = per_w // (2 * C)
    LV = LD // 16

    mesh = plsc.VectorSubcoreMesh(core_axis_name="c", subcore_axis_name="s")

    buf = lambda shape, dt: pltpu.VMEM(shape, dt)

    @functools.partial(
        pl.kernel,
        out_type=jax.ShapeDtypeStruct((N, TD + NSETS * LD), jnp.float32),
        mesh=mesh,
        compiler_params=pltpu.CompilerParams(use_tc_tiling_on_sc=False),
        scratch_types=[
            buf((C,), jnp.int32), buf((C,), jnp.int32),
            buf((C * NL,), jnp.int32), buf((C * NL,), jnp.int32),
            buf((C * 32,), jnp.float32), buf((C * 32,), jnp.float32),
            buf((C, TD), jnp.float32), buf((C, TD), jnp.float32),
            buf((C * NL, LD), jnp.float32), buf((C * NL, LD), jnp.float32),
            buf((C, NSETS * LD), jnp.float32),
            pltpu.SemaphoreType.DMA, pltpu.SemaphoreType.DMA,
        ],
    )
    def sc_kernel(tok_ids, lex_ids, w_flat, tok_tab, lex_tab, out,
                  tok_idx_a, tok_idx_b, lex_idx_a, lex_idx_b,
                  w_a, w_b, tok_rows_a, tok_rows_b,
                  lex_rows_a, lex_rows_b, lex_out, sem_a, sem_b):
        nc = 2
        wid = lax.axis_index("s") * nc + lax.axis_index("c")

        def fire(base, tok_idx_v, lex_idx_v, w_v, tok_rows, lex_rows, sem):
            pltpu.sync_copy(tok_ids.at[pl.ds(base, C)], tok_idx_v)
            pltpu.sync_copy(lex_ids.at[pl.ds(base * NL, C * NL)], lex_idx_v)
            pltpu.sync_copy(w_flat.at[pl.ds(base * 32, C * 32)], w_v)
            cps = [pltpu.async_copy(tok_tab.at[tok_idx_v], tok_rows, sem),
                   pltpu.async_copy(lex_tab.at[lex_idx_v], lex_rows, sem)]
            return cps

        def finish(base, cps, w_v, tok_rows, lex_rows):
            for cp in cps:
                cp.wait()

            def pos_body(i, c2):
                rbase = i * NL
                wrows = [w_v[pl.ds(i * 32, 16)], w_v[pl.ds(i * 32 + 16, 16)]]
                for n in range(NSETS):
                    acc = [jnp.zeros((16,), jnp.float32) for _ in range(LV)]
                    for m in range(MAXLEX):
                        r = rbase + n * MAXLEX + m
                        k = n * MAXLEX + m
                        wv = wrows[k // 16][k % 16]
                        for cc in range(LV):
                            acc[cc] = acc[cc] + lex_rows[r, pl.ds(cc * 16, 16)] * wv
                    for cc in range(LV):
                        lex_out[i, pl.ds(n * LD + cc * 16, 16)] = acc[cc]
                return c2

            lax.fori_loop(0, C, pos_body, 0)

            pltpu.sync_copy(tok_rows, out.at[pl.ds(base, C), pl.ds(0, TD)])
            pltpu.sync_copy(
                lex_out, out.at[pl.ds(base, C), pl.ds(TD, NSETS * LD)])

        def pair_body(k, carry):
            base_a = wid * per_w + (2 * k) * C
            base_b = base_a + C
            cps_a = fire(base_a, tok_idx_a, lex_idx_a, w_a,
                         tok_rows_a, lex_rows_a, sem_a)
            cps_b = fire(base_b, tok_idx_b, lex_idx_b, w_b,
                         tok_rows_b, lex_rows_b, sem_b)
            finish(base_a, cps_a, w_a, tok_rows_a, lex_rows_a)
            finish(base_b, cps_b, w_b, tok_rows_b, lex_rows_b)
            return carry

        lax.fori_loop(0, n_pairs, pair_body, 0)

    return sc_kernel


def kernel(inputs, lexicons, weights, token_table, lexicon_table):
    B, S = inputs.shape
    _, _, NSETS, MAXLEX = lexicons.shape
    TD = token_table.shape[1]
    LD = lexicon_table.shape[1]
    N = B * S
    NL = NSETS * MAXLEX
    NW = 32
    C = 32

    tok_ids = inputs.reshape(N).astype(jnp.int32)
    lex_ids = lexicons.reshape(N * NL).astype(jnp.int32)
    w_flat = jnp.pad(weights.reshape(N, NL), ((0, 0), (0, 32 - NL))).reshape(N * 32)

    sc = _build_sc_kernel(N, TD, LD, NSETS, MAXLEX, NW, C)
    out = sc(tok_ids, lex_ids, w_flat, token_table, lexicon_table)
    return out.reshape(B, S, TD + NSETS * LD)


# speedup vs baseline: 1.0008x; 1.0008x over previous
import functools

import jax
import jax.numpy as jnp
from jax import lax
from jax.experimental import pallas as pl
from jax.experimental.pallas import tpu as pltpu
from jax.experimental.pallas import tpu_sc as plsc


def _build_sc_kernel(N, TD, LD, NSETS, MAXLEX, NW, C):
    NL = NSETS * MAXLEX
    per_w = N // NW
    assert per_w % (2 * C) == 0
    n_pairs = per_w // (2 * C)
    LV = LD // 16

    mesh = plsc.VectorSubcoreMesh(core_axis_name="c", subcore_axis_name="s")

    buf = lambda shape, dt: pltpu.VMEM(shape, dt)

    @functools.partial(
        pl.kernel,
        out_type=jax.ShapeDtypeStruct((N, TD + NSETS * LD), jnp.float32),
        mesh=mesh,
        compiler_params=pltpu.CompilerParams(use_tc_tiling_on_sc=False),
        scratch_types=[
            buf((C,), jnp.int32), buf((C,), jnp.int32),
            buf((C * NL,), jnp.int32), buf((C * NL,), jnp.int32),
            buf((C * 32,), jnp.float32), buf((C * 32,), jnp.float32),
            buf((C, TD), jnp.float32), buf((C, TD), jnp.float32),
            buf((C * NL, LD), jnp.float32), buf((C * NL, LD), jnp.float32),
            buf((C, NSETS * LD), jnp.float32),
            pltpu.SemaphoreType.DMA, pltpu.SemaphoreType.DMA,
        ],
    )
    def sc_kernel(tok_ids, lex_ids, w_flat, tok_tab, lex_tab, out,
                  tok_idx_a, tok_idx_b, lex_idx_a, lex_idx_b,
                  w_a, w_b, tok_rows_a, tok_rows_b,
                  lex_rows_a, lex_rows_b, lex_out, sem_a, sem_b):
        nc = 2
        wid = lax.axis_index("s") * nc + lax.axis_index("c")

        def fire(base, tok_idx_v, lex_idx_v, w_v, tok_rows, lex_rows, sem):
            pltpu.sync_copy(tok_ids.at[pl.ds(base, C)], tok_idx_v)
            pltpu.sync_copy(lex_ids.at[pl.ds(base * NL, C * NL)], lex_idx_v)
            pltpu.sync_copy(w_flat.at[pl.ds(base * 32, C * 32)], w_v)
            cps = [pltpu.async_copy(tok_tab.at[tok_idx_v], tok_rows, sem),
                   pltpu.async_copy(lex_tab.at[lex_idx_v], lex_rows, sem)]
            return cps

        def finish(base, cps, w_v, tok_rows, lex_rows):
            for cp in cps:
                cp.wait()

            def pos_body(i, c2):
                rbase = i * NL
                wrows = [w_v[pl.ds(i * 32, 16)], w_v[pl.ds(i * 32 + 16, 16)]]
                for n in range(NSETS):
                    acc = [jnp.zeros((16,), jnp.float32) for _ in range(LV)]
                    for m in range(MAXLEX):
                        r = rbase + n * MAXLEX + m
                        k = n * MAXLEX + m
                        wv = wrows[k // 16][k % 16]
                        for cc in range(LV):
                            acc[cc] = acc[cc] + lex_rows[r, pl.ds(cc * 16, 16)] * wv
                    for cc in range(LV):
                        lex_out[i, pl.ds(n * LD + cc * 16, 16)] = acc[cc]
                return c2

            lax.fori_loop(0, C, pos_body, 0)

            pltpu.sync_copy(tok_rows, out.at[pl.ds(base, C), pl.ds(0, TD)])
            pltpu.sync_copy(
                lex_out, out.at[pl.ds(base, C), pl.ds(TD, NSETS * LD)])

        def pair_body(k, carry):
            base_a = wid * per_w + (2 * k) * C
            base_b = base_a + C
            cps_a = fire(base_a, tok_idx_a, lex_idx_a, w_a,
                         tok_rows_a, lex_rows_a, sem_a)
            cps_b = fire(base_b, tok_idx_b, lex_idx_b, w_b,
                         tok_rows_b, lex_rows_b, sem_b)
            finish(base_a, cps_a, w_a, tok_rows_a, lex_rows_a)
            finish(base_b, cps_b, w_b, tok_rows_b, lex_rows_b)
            return carry

        lax.fori_loop(0, n_pairs, pair_body, 0)

    return sc_kernel


def kernel(inputs, lexicons, weights, token_table, lexicon_table):
    B, S = inputs.shape
    _, _, NSETS, MAXLEX = lexicons.shape
    TD = token_table.shape[1]
    LD = lexicon_table.shape[1]
    N = B * S
    NL = NSETS * MAXLEX
    NW = 32
    C = 32

    tok_ids = inputs.reshape(N).astype(jnp.int32)
    lex_ids = lexicons.reshape(N * NL).astype(jnp.int32)
    w_flat = jnp.pad(weights.reshape(N, NL), ((0, 0), (0, 32 - NL))).reshape(N * 32)

    sc = _build_sc_kernel(N, TD, LD, NSETS, MAXLEX, NW, C)
    out = sc(tok_ids, lex_ids, w_flat, token_table, lexicon_table)
    return out.reshape(B, S, TD + NSETS * LD)
